# baseline (speedup 1.0000x reference)
import functools

import jax
import jax.numpy as jnp
from jax import lax
from jax.experimental import pallas as pl
from jax.experimental.pallas import tpu as pltpu
from jax.experimental.pallas import tpu_sc as plsc

_NUM_CLASSES = 1000000
_FEAT = 64
_BATCH = 16384
_LAMBDA_C = 0.001

_NC = 2
_NS = 16
_NW = _NC * _NS
_RW = 128
_ROWS = _BATCH * _FEAT // _NW // _RW
_WAVE = 8
_L = 16


def _partials_kernel(feat_hbm, idx_hbm, cent_hbm, out_hbm,
                     idx_v, feat_v, acc_v, sem, fsem, *slots):
    wid = lax.axis_index("s") * _NC + lax.axis_index("c")

    pltpu.sync_copy(idx_hbm.at[wid], idx_v)
    pltpu.async_copy(feat_hbm.at[wid], feat_v, fsem).wait()

    def wave(g, accs):
        copies = [
            pltpu.async_copy(cent_hbm.at[idx_v.at[g * _WAVE + u]],
                             slots[u], sem)
            for u in range(_WAVE)
        ]
        accs = list(accs)
        for u in range(_WAVE):
            copies[u].wait()
            r = g * _WAVE + u
            for c in range(_RW // _L):
                x = feat_v[r, pl.ds(c * _L, _L)]
                y = slots[u][pl.ds(c * _L, _L)]
                d = x - y
                accs[c % 4] += d * d
        return tuple(accs)

    zero = jnp.zeros((_L,), jnp.float32)
    accs = lax.fori_loop(0, _ROWS // _WAVE, wave, (zero,) * 4)
    acc_v[...] = (accs[0] + accs[1]) + (accs[2] + accs[3])
    pltpu.sync_copy(acc_v, out_hbm.at[wid])


@functools.partial(
    pl.kernel,
    mesh=plsc.VectorSubcoreMesh(core_axis_name="c", subcore_axis_name="s"),
    out_type=jax.ShapeDtypeStruct((_NW, _L), jnp.float32),
    scratch_types=[
        pltpu.VMEM((_ROWS, _RW), jnp.int32),
        pltpu.VMEM((_ROWS, _RW), jnp.float32),
        pltpu.VMEM((_L,), jnp.float32),
        pltpu.SemaphoreType.DMA,
        pltpu.SemaphoreType.DMA,
    ] + [pltpu.VMEM((_RW,), jnp.float32)] * _WAVE,
)
def _partials(feat_hbm, idx_hbm, cent_hbm, out_hbm,
              idx_v, feat_v, acc_v, sem, fsem, *slots):
    _partials_kernel(feat_hbm, idx_hbm, cent_hbm, out_hbm,
                     idx_v, feat_v, acc_v, sem, fsem, *slots)


def kernel(features, labels, centers):
    cent_flat = centers.T.reshape(-1)
    lab = labels.astype(jnp.int32)
    offs = jnp.arange(_FEAT, dtype=jnp.int32) * _NUM_CLASSES
    idx = (lab[:, None] + offs[None, :]).reshape(_NW, _ROWS, _RW)
    feat = features.reshape(_NW, _ROWS, _RW)
    partials = _partials(feat, idx, cent_flat)
    return (_LAMBDA_C * 0.5 / _BATCH) * jnp.sum(partials)


# speedup vs baseline: 1.0061x; 1.0061x over previous
import functools

import jax
import jax.numpy as jnp
from jax import lax
from jax.experimental import pallas as pl
from jax.experimental.pallas import tpu as pltpu
from jax.experimental.pallas import tpu_sc as plsc

_NUM_CLASSES = 1000000
_FEAT = 64
_BATCH = 16384
_LAMBDA_C = 0.001

_NC = 2
_NS = 16
_NW = _NC * _NS
_RW = 2048
_ROWS = _BATCH * _FEAT // _NW // _RW
_WAVE = 4
_L = 16


def _partials_kernel(feat_hbm, idx_hbm, cent_hbm, out_hbm,
                     idx_v, feat_v, acc_v, sem, fsem, *slots):
    wid = lax.axis_index("s") * _NC + lax.axis_index("c")

    pltpu.sync_copy(idx_hbm.at[wid], idx_v)
    pltpu.async_copy(feat_hbm.at[wid], feat_v, fsem).wait()

    def wave(g, accs):
        copies = [
            pltpu.async_copy(
                cent_hbm.at[idx_v.at[pl.ds((g * _WAVE + u) * _RW, _RW)]],
                slots[u], sem)
            for u in range(_WAVE)
        ]
        accs = list(accs)
        for u in range(_WAVE):
            copies[u].wait()
            r = g * _WAVE + u
            for c in range(_RW // _L):
                x = feat_v[r, pl.ds(c * _L, _L)]
                y = slots[u][pl.ds(c * _L, _L)]
                d = x - y
                accs[c % 4] += d * d
        return tuple(accs)

    zero = jnp.zeros((_L,), jnp.float32)
    accs = lax.fori_loop(0, _ROWS // _WAVE, wave, (zero,) * 4)
    acc_v[...] = (accs[0] + accs[1]) + (accs[2] + accs[3])
    pltpu.sync_copy(acc_v, out_hbm.at[wid])


@functools.partial(
    pl.kernel,
    mesh=plsc.VectorSubcoreMesh(core_axis_name="c", subcore_axis_name="s"),
    out_type=jax.ShapeDtypeStruct((_NW, _L), jnp.float32),
    scratch_types=[
        pltpu.VMEM((_ROWS * _RW,), jnp.int32),
        pltpu.VMEM((_ROWS, _RW), jnp.float32),
        pltpu.VMEM((_L,), jnp.float32),
        pltpu.SemaphoreType.DMA,
        pltpu.SemaphoreType.DMA,
    ] + [pltpu.VMEM((_RW,), jnp.float32)] * _WAVE,
)
def _partials(feat_hbm, idx_hbm, cent_hbm, out_hbm,
              idx_v, feat_v, acc_v, sem, fsem, *slots):
    _partials_kernel(feat_hbm, idx_hbm, cent_hbm, out_hbm,
                     idx_v, feat_v, acc_v, sem, fsem, *slots)


def kernel(features, labels, centers):
    cent_flat = centers.T.reshape(-1)
    lab = labels.astype(jnp.int32)
    offs = jnp.arange(_FEAT, dtype=jnp.int32) * _NUM_CLASSES
    idx = (lab[:, None] + offs[None, :]).reshape(_NW, _ROWS * _RW)
    feat = features.reshape(_NW, _ROWS, _RW)
    partials = _partials(feat, idx, cent_flat)
    return (_LAMBDA_C * 0.5 / _BATCH) * jnp.sum(partials)
